# baseline (speedup 1.0000x reference)
import functools

import jax
import jax.numpy as jnp
from jax import lax
from jax.experimental import pallas as pl
from jax.experimental.pallas import tpu as pltpu
from jax.experimental.pallas import tpu_sc as plsc

_NC, _NS = 2, 16
_NW = _NC * _NS
_K = 4
_SLOTS = 4
_BB = 8


def _chunk_size(n_w):
    for ch in (128, 120, 112, 104, 96, 88, 80, 72, 64, 56, 48, 40, 32):
        if n_w % ch == 0:
            return ch
    return 8


def _sc_gather(table, idx_flat, chunk_base, n_rows):
    v_dim, e_dim = table.shape
    n_w = n_rows // _NW
    ch = _chunk_size(n_w)
    n_ch = n_w // ch

    mesh = plsc.VectorSubcoreMesh(
        core_axis_name="c", subcore_axis_name="s",
        num_cores=_NC, num_subcores=_NS)

    @functools.partial(
        pl.kernel,
        out_type=jax.ShapeDtypeStruct((n_rows, e_dim), jnp.float32),
        mesh=mesh,
        compiler_params=pltpu.CompilerParams(
            use_tc_tiling_on_sc=False, needs_layout_passes=False),
        scratch_types=[
            pltpu.VMEM((n_w,), jnp.int32),
            pltpu.VMEM((_SLOTS, ch, 128), jnp.float32),
            [pltpu.SemaphoreType.DMA] * _SLOTS,
            [pltpu.SemaphoreType.DMA] * _SLOTS,
        ],
    )
    def sc_k(tab_hbm, idx_hbm, out_hbm, idx_v, buf_v, gsems, wsems):
        wid = lax.axis_index("s") * _NC + lax.axis_index("c")
        base = wid * n_w
        pltpu.sync_copy(idx_hbm.at[pl.ds(chunk_base + base, n_w)], idx_v)

        def fire(c):
            s = c % _SLOTS
            return pltpu.async_copy(
                tab_hbm.at[idx_v.at[pl.ds(c * ch, ch)]],
                buf_v.at[s], gsems[s])

        gathers = [fire(c) for c in range(min(_SLOTS, n_ch))]
        writes = [None] * n_ch
        for c in range(n_ch):
            s = c % _SLOTS
            gathers[c].wait()
            writes[c] = pltpu.async_copy(
                buf_v.at[s], out_hbm.at[pl.ds(base + c * ch, ch)], wsems[s])
            nxt = c + _SLOTS
            if nxt < n_ch:
                writes[c].wait()
                gathers.append(fire(nxt))
        for c in range(max(0, n_ch - _SLOTS), n_ch):
            writes[c].wait()

    return sc_k(table, idx_flat)


def _tc_body_first(seq_ref, emb_ref, out_ref):
    bb, _, l_dim = seq_ref.shape
    l2 = 2 * l_dim
    rows = jax.lax.broadcasted_iota(jnp.int32, (l_dim, l2), 0)
    cols = jax.lax.broadcasted_iota(jnp.int32, (l_dim, l2), 1)
    perm = (cols == rows + l_dim).astype(jnp.bfloat16)
    for b in range(bb):
        emb_pad = jax.lax.dot_general(
            emb_ref[b].astype(jnp.bfloat16), perm, (((0,), (0,)), ((), ())),
            preferred_element_type=jnp.float32)
        out_ref[b] = emb_pad
        out_ref[b, :, pl.ds(0, l_dim)] = seq_ref[b]


def _tc_body_chained(seq_ref, emb_ref, prev_ref, out_ref):
    del prev_ref
    _tc_body_first(seq_ref, emb_ref, out_ref)


def _tc_merge_chunk(sequence, emb_chunk, prev_out, chunk_idx, bc):
    b_total, e_dim, l_dim = sequence.shape
    base_blk = chunk_idx * (bc // _BB)
    seq_spec = pl.BlockSpec((_BB, e_dim, l_dim),
                            lambda i: (base_blk + i, 0, 0))
    emb_spec = pl.BlockSpec((_BB, l_dim, e_dim), lambda i: (i, 0, 0))
    out_spec = pl.BlockSpec((_BB, e_dim, 2 * l_dim),
                            lambda i: (base_blk + i, 0, 0))
    out_shape = jax.ShapeDtypeStruct((b_total, e_dim, 2 * l_dim), jnp.float32)
    if prev_out is None:
        return pl.pallas_call(
            _tc_body_first,
            grid=(bc // _BB,),
            in_specs=[seq_spec, emb_spec],
            out_specs=out_spec,
            out_shape=out_shape,
        )(sequence, emb_chunk)
    return pl.pallas_call(
        _tc_body_chained,
        grid=(bc // _BB,),
        in_specs=[seq_spec, emb_spec, pl.BlockSpec(memory_space=pl.ANY)],
        out_specs=out_spec,
        out_shape=out_shape,
        input_output_aliases={2: 0},
    )(sequence, emb_chunk, prev_out)


def kernel(sequence, time_index_sequence, variable_index_sequence, embed_table):
    del time_index_sequence
    b_total, e_dim, l_dim = sequence.shape
    idx_flat = variable_index_sequence.reshape(b_total * l_dim).astype(jnp.int32)
    bc = b_total // _K
    nc = bc * l_dim
    emb_chunks = [
        _sc_gather(embed_table, idx_flat, k * nc, nc) for k in range(_K)
    ]
    out = None
    for k in range(_K):
        emb = emb_chunks[k].reshape(bc, l_dim, e_dim)
        out = _tc_merge_chunk(sequence, emb, out, k, bc)
    return out


# speedup vs baseline: 2.4803x; 2.4803x over previous
import functools

import jax
import jax.numpy as jnp
from jax import lax
from jax.experimental import pallas as pl
from jax.experimental.pallas import tpu as pltpu
from jax.experimental.pallas import tpu_sc as plsc

_NC, _NS = 2, 16
_NW = _NC * _NS
_SLOTS = 4
_BB = 8


def _sc_fill_right(table, idx2d, b_total, l_dim):
    v_dim, e_dim = table.shape
    lh = l_dim // 2
    n_b = b_total // _NW
    n_ch = 2 * n_b

    mesh = plsc.VectorSubcoreMesh(
        core_axis_name="c", subcore_axis_name="s",
        num_cores=_NC, num_subcores=_NS)

    @functools.partial(
        pl.kernel,
        out_type=jax.ShapeDtypeStruct((b_total, 2 * l_dim, e_dim),
                                      jnp.float32),
        mesh=mesh,
        compiler_params=pltpu.CompilerParams(
            use_tc_tiling_on_sc=False, needs_layout_passes=False),
        scratch_types=[
            pltpu.VMEM((n_ch, lh), jnp.int32),
            pltpu.VMEM((_SLOTS, lh, 128), jnp.float32),
            [pltpu.SemaphoreType.DMA] * _SLOTS,
            [pltpu.SemaphoreType.DMA] * _SLOTS,
        ],
    )
    def sc_k(tab_hbm, idx_hbm, out_hbm, idx_v, buf_v, gsems, wsems):
        wid = lax.axis_index("s") * _NC + lax.axis_index("c")
        pltpu.sync_copy(idx_hbm.at[pl.ds(wid * n_ch, n_ch)], idx_v)

        def fire(c):
            s = c % _SLOTS
            return pltpu.async_copy(
                tab_hbm.at[idx_v.at[c]], buf_v.at[s], gsems[s])

        gathers = [fire(c) for c in range(min(_SLOTS, n_ch))]
        writes = [None] * n_ch
        for c in range(n_ch):
            s = c % _SLOTS
            gathers[c].wait()
            writes[c] = pltpu.async_copy(
                buf_v.at[s],
                out_hbm.at[wid * n_b + c // 2,
                           pl.ds(l_dim + (c % 2) * lh, lh), :],
                wsems[s])
            nxt = c + _SLOTS
            if nxt < n_ch:
                writes[c].wait()
                gathers.append(fire(nxt))
        for c in range(max(0, n_ch - _SLOTS), n_ch):
            writes[c].wait()

    return sc_k(table, idx2d)


def _tc_body(seq_ref, prev_ref, out_ref):
    del prev_ref
    out_ref[...] = seq_ref[...]


def _tc_fill_left(seq_t, out_t):
    b_total, l2, e_dim = out_t.shape
    l_dim = l2 // 2
    return pl.pallas_call(
        _tc_body,
        grid=(b_total // _BB,),
        in_specs=[
            pl.BlockSpec((_BB, l_dim, e_dim), lambda i: (i, 0, 0)),
            pl.BlockSpec(memory_space=pl.ANY),
        ],
        out_specs=pl.BlockSpec((_BB, l_dim, e_dim), lambda i: (i, 0, 0)),
        out_shape=jax.ShapeDtypeStruct((b_total, l2, e_dim), jnp.float32),
        input_output_aliases={1: 0},
    )(seq_t, out_t)


def kernel(sequence, time_index_sequence, variable_index_sequence, embed_table):
    del time_index_sequence
    b_total, e_dim, l_dim = sequence.shape
    seq_t = jnp.transpose(sequence, (0, 2, 1))
    idx2d = variable_index_sequence.reshape(
        2 * b_total, l_dim // 2).astype(jnp.int32)
    out_t = _sc_fill_right(embed_table, idx2d, b_total, l_dim)
    out_t = _tc_fill_left(seq_t, out_t)
    return jnp.transpose(out_t, (0, 2, 1))


# speedup vs baseline: 2.8658x; 1.1554x over previous
import functools

import jax
import jax.numpy as jnp
from jax import lax
from jax.experimental import pallas as pl
from jax.experimental.pallas import tpu as pltpu
from jax.experimental.pallas import tpu_sc as plsc

_NC, _NS = 2, 16
_NW = _NC * _NS
_SLOTS = 6
_BB = 16


def _sc_fill_right(table, idx2d, b_total, l_dim):
    v_dim, e_dim = table.shape
    lh = l_dim // 2
    n_b = b_total // _NW
    n_ch = 2 * n_b

    mesh = plsc.VectorSubcoreMesh(
        core_axis_name="c", subcore_axis_name="s",
        num_cores=_NC, num_subcores=_NS)

    @functools.partial(
        pl.kernel,
        out_type=jax.ShapeDtypeStruct((b_total, 2 * l_dim, e_dim),
                                      jnp.float32),
        mesh=mesh,
        compiler_params=pltpu.CompilerParams(
            use_tc_tiling_on_sc=False, needs_layout_passes=False),
        scratch_types=[
            pltpu.VMEM((n_ch, lh), jnp.int32),
            pltpu.VMEM((_SLOTS, lh, 128), jnp.float32),
            [pltpu.SemaphoreType.DMA] * _SLOTS,
            [pltpu.SemaphoreType.DMA] * _SLOTS,
        ],
    )
    def sc_k(tab_hbm, idx_hbm, out_hbm, idx_v, buf_v, gsems, wsems):
        wid = lax.axis_index("s") * _NC + lax.axis_index("c")
        pltpu.sync_copy(idx_hbm.at[pl.ds(wid * n_ch, n_ch)], idx_v)

        def fire(c):
            s = c % _SLOTS
            return pltpu.async_copy(
                tab_hbm.at[idx_v.at[c]], buf_v.at[s], gsems[s])

        gathers = [fire(c) for c in range(min(_SLOTS, n_ch))]
        writes = [None] * n_ch
        for c in range(n_ch):
            s = c % _SLOTS
            gathers[c].wait()
            writes[c] = pltpu.async_copy(
                buf_v.at[s],
                out_hbm.at[wid * n_b + c // 2,
                           pl.ds(l_dim + (c % 2) * lh, lh), :],
                wsems[s])
            nxt = c + _SLOTS
            if nxt < n_ch:
                writes[c].wait()
                gathers.append(fire(nxt))
        for c in range(max(0, n_ch - _SLOTS), n_ch):
            writes[c].wait()

    return sc_k(table, idx2d)


def _tc_body(seq_ref, prev_ref, out_ref):
    del prev_ref
    out_ref[...] = seq_ref[...]


def _tc_fill_left(seq_t, out_t):
    b_total, l2, e_dim = out_t.shape
    l_dim = l2 // 2
    return pl.pallas_call(
        _tc_body,
        grid=(b_total // _BB,),
        in_specs=[
            pl.BlockSpec((_BB, l_dim, e_dim), lambda i: (i, 0, 0)),
            pl.BlockSpec(memory_space=pl.ANY),
        ],
        out_specs=pl.BlockSpec((_BB, l_dim, e_dim), lambda i: (i, 0, 0)),
        out_shape=jax.ShapeDtypeStruct((b_total, l2, e_dim), jnp.float32),
        input_output_aliases={1: 0},
    )(seq_t, out_t)


def kernel(sequence, time_index_sequence, variable_index_sequence, embed_table):
    del time_index_sequence
    b_total, e_dim, l_dim = sequence.shape
    seq_t = jnp.transpose(sequence, (0, 2, 1))
    idx2d = variable_index_sequence.reshape(
        2 * b_total, l_dim // 2).astype(jnp.int32)
    out_t = _sc_fill_right(embed_table, idx2d, b_total, l_dim)
    out_t = _tc_fill_left(seq_t, out_t)
    return jnp.transpose(out_t, (0, 2, 1))


# speedup vs baseline: 3.0281x; 1.0566x over previous
import functools

import jax
import jax.numpy as jnp
from jax import lax
from jax.experimental import pallas as pl
from jax.experimental.pallas import tpu as pltpu
from jax.experimental.pallas import tpu_sc as plsc

_NC, _NS = 2, 16
_NW = _NC * _NS
_SLOTS = 8
_BB = 32


def _sc_fill_right(table, idx2d, b_total, l_dim):
    v_dim, e_dim = table.shape
    lh = l_dim // 2
    n_b = b_total // _NW
    n_ch = 2 * n_b

    mesh = plsc.VectorSubcoreMesh(
        core_axis_name="c", subcore_axis_name="s",
        num_cores=_NC, num_subcores=_NS)

    @functools.partial(
        pl.kernel,
        out_type=jax.ShapeDtypeStruct((b_total, 2 * l_dim, e_dim),
                                      jnp.float32),
        mesh=mesh,
        compiler_params=pltpu.CompilerParams(
            use_tc_tiling_on_sc=False, needs_layout_passes=False),
        scratch_types=[
            pltpu.VMEM((n_ch, lh), jnp.int32),
            pltpu.VMEM((_SLOTS, lh, 128), jnp.float32),
            [pltpu.SemaphoreType.DMA] * _SLOTS,
            [pltpu.SemaphoreType.DMA] * _SLOTS,
        ],
    )
    def sc_k(tab_hbm, idx_hbm, out_hbm, idx_v, buf_v, gsems, wsems):
        wid = lax.axis_index("s") * _NC + lax.axis_index("c")
        pltpu.sync_copy(idx_hbm.at[pl.ds(wid * n_ch, n_ch)], idx_v)

        def fire(c):
            s = c % _SLOTS
            return pltpu.async_copy(
                tab_hbm.at[idx_v.at[c]], buf_v.at[s], gsems[s])

        gathers = [fire(c) for c in range(min(_SLOTS, n_ch))]
        writes = [None] * n_ch
        for c in range(n_ch):
            s = c % _SLOTS
            gathers[c].wait()
            writes[c] = pltpu.async_copy(
                buf_v.at[s],
                out_hbm.at[wid * n_b + c // 2,
                           pl.ds(l_dim + (c % 2) * lh, lh), :],
                wsems[s])
            nxt = c + _SLOTS
            if nxt < n_ch:
                writes[c].wait()
                gathers.append(fire(nxt))
        for c in range(max(0, n_ch - _SLOTS), n_ch):
            writes[c].wait()

    return sc_k(table, idx2d)


def _tc_body(seq_ref, prev_ref, out_ref):
    del prev_ref
    out_ref[...] = seq_ref[...]


def _tc_fill_left(seq_t, out_t):
    b_total, l2, e_dim = out_t.shape
    l_dim = l2 // 2
    return pl.pallas_call(
        _tc_body,
        grid=(b_total // _BB,),
        in_specs=[
            pl.BlockSpec((_BB, l_dim, e_dim), lambda i: (i, 0, 0)),
            pl.BlockSpec(memory_space=pl.ANY),
        ],
        out_specs=pl.BlockSpec((_BB, l_dim, e_dim), lambda i: (i, 0, 0)),
        out_shape=jax.ShapeDtypeStruct((b_total, l2, e_dim), jnp.float32),
        input_output_aliases={1: 0},
    )(seq_t, out_t)


def kernel(sequence, time_index_sequence, variable_index_sequence, embed_table):
    del time_index_sequence
    b_total, e_dim, l_dim = sequence.shape
    seq_t = jnp.transpose(sequence, (0, 2, 1))
    idx2d = variable_index_sequence.reshape(
        2 * b_total, l_dim // 2).astype(jnp.int32)
    out_t = _sc_fill_right(embed_table, idx2d, b_total, l_dim)
    out_t = _tc_fill_left(seq_t, out_t)
    return jnp.transpose(out_t, (0, 2, 1))


# speedup vs baseline: 3.0564x; 1.0094x over previous
import functools

import jax
import jax.numpy as jnp
from jax import lax
from jax.experimental import pallas as pl
from jax.experimental.pallas import tpu as pltpu
from jax.experimental.pallas import tpu_sc as plsc

_NC, _NS = 2, 16
_NW = _NC * _NS
_SLOTS = 8
_BB = 64


def _sc_fill_right(table, idx2d, b_total, l_dim):
    v_dim, e_dim = table.shape
    lh = l_dim // 2
    n_b = b_total // _NW
    n_ch = 2 * n_b

    mesh = plsc.VectorSubcoreMesh(
        core_axis_name="c", subcore_axis_name="s",
        num_cores=_NC, num_subcores=_NS)

    @functools.partial(
        pl.kernel,
        out_type=jax.ShapeDtypeStruct((b_total, 2 * l_dim, e_dim),
                                      jnp.float32),
        mesh=mesh,
        compiler_params=pltpu.CompilerParams(
            use_tc_tiling_on_sc=False, needs_layout_passes=False),
        scratch_types=[
            pltpu.VMEM((n_ch, lh), jnp.int32),
            pltpu.VMEM((_SLOTS, lh, 128), jnp.float32),
            [pltpu.SemaphoreType.DMA] * _SLOTS,
            [pltpu.SemaphoreType.DMA] * _SLOTS,
        ],
    )
    def sc_k(tab_hbm, idx_hbm, out_hbm, idx_v, buf_v, gsems, wsems):
        wid = lax.axis_index("s") * _NC + lax.axis_index("c")
        pltpu.sync_copy(idx_hbm.at[pl.ds(wid * n_ch, n_ch)], idx_v)

        def fire(c):
            s = c % _SLOTS
            return pltpu.async_copy(
                tab_hbm.at[idx_v.at[c]], buf_v.at[s], gsems[s])

        gathers = [fire(c) for c in range(min(_SLOTS, n_ch))]
        writes = [None] * n_ch
        for c in range(n_ch):
            s = c % _SLOTS
            gathers[c].wait()
            writes[c] = pltpu.async_copy(
                buf_v.at[s],
                out_hbm.at[wid * n_b + c // 2,
                           pl.ds(l_dim + (c % 2) * lh, lh), :],
                wsems[s])
            nxt = c + _SLOTS
            if nxt < n_ch:
                writes[c].wait()
                gathers.append(fire(nxt))
        for c in range(max(0, n_ch - _SLOTS), n_ch):
            writes[c].wait()

    return sc_k(table, idx2d)


def _tc_body(seq_ref, prev_ref, out_ref):
    del prev_ref
    out_ref[...] = seq_ref[...]


def _tc_fill_left(seq_t, out_t):
    b_total, l2, e_dim = out_t.shape
    l_dim = l2 // 2
    return pl.pallas_call(
        _tc_body,
        grid=(b_total // _BB,),
        in_specs=[
            pl.BlockSpec((_BB, l_dim, e_dim), lambda i: (i, 0, 0)),
            pl.BlockSpec(memory_space=pl.ANY),
        ],
        out_specs=pl.BlockSpec((_BB, l_dim, e_dim), lambda i: (i, 0, 0)),
        out_shape=jax.ShapeDtypeStruct((b_total, l2, e_dim), jnp.float32),
        input_output_aliases={1: 0},
    )(seq_t, out_t)


def kernel(sequence, time_index_sequence, variable_index_sequence, embed_table):
    del time_index_sequence
    b_total, e_dim, l_dim = sequence.shape
    seq_t = jnp.transpose(sequence, (0, 2, 1))
    idx2d = variable_index_sequence.reshape(
        2 * b_total, l_dim // 2).astype(jnp.int32)
    out_t = _sc_fill_right(embed_table, idx2d, b_total, l_dim)
    out_t = _tc_fill_left(seq_t, out_t)
    return jnp.transpose(out_t, (0, 2, 1))


# speedup vs baseline: 3.0633x; 1.0023x over previous
import functools

import jax
import jax.numpy as jnp
from jax import lax
from jax.experimental import pallas as pl
from jax.experimental.pallas import tpu as pltpu
from jax.experimental.pallas import tpu_sc as plsc

_NC, _NS = 2, 16
_NW = _NC * _NS
_SLOTS = 8
_BB = 64


def _sc_fill_right(table, idx2d, b_total, l_dim):
    e_dim = table.shape[1]
    lh = l_dim // 2
    n_b = b_total // _NW
    n_ch = 2 * n_b

    mesh = plsc.VectorSubcoreMesh(
        core_axis_name="c", subcore_axis_name="s",
        num_cores=_NC, num_subcores=_NS)

    @functools.partial(
        pl.kernel,
        out_type=jax.ShapeDtypeStruct((b_total, 2 * l_dim, e_dim),
                                      jnp.float32),
        mesh=mesh,
        compiler_params=pltpu.CompilerParams(
            use_tc_tiling_on_sc=False, needs_layout_passes=False),
        scratch_types=[
            pltpu.VMEM((n_ch, lh), jnp.int32),
            pltpu.VMEM((_SLOTS, lh, 128), jnp.float32),
            [pltpu.SemaphoreType.DMA] * _SLOTS,
            [pltpu.SemaphoreType.DMA] * _SLOTS,
        ],
    )
    def sc_k(tab_hbm, idx_hbm, out_hbm, idx_v, buf_v, gsems, wsems):
        wid = lax.axis_index("s") * _NC + lax.axis_index("c")
        pltpu.sync_copy(idx_hbm.at[pl.ds(wid * n_ch, n_ch)], idx_v)

        def fire(c):
            s = c % _SLOTS
            return pltpu.async_copy(
                tab_hbm.at[idx_v.at[c]], buf_v.at[s], gsems[s])

        gathers = [fire(c) for c in range(min(_SLOTS, n_ch))]
        writes = [None] * n_ch
        for c in range(n_ch):
            s = c % _SLOTS
            gathers[c].wait()
            writes[c] = pltpu.async_copy(
                buf_v.at[s],
                out_hbm.at[wid * n_b + c // 2,
                           pl.ds(l_dim + (c % 2) * lh, lh), :],
                wsems[s])
            nxt = c + _SLOTS
            if nxt < n_ch:
                writes[c].wait()
                gathers.append(fire(nxt))
        for c in range(max(0, n_ch - _SLOTS), n_ch):
            writes[c].wait()

    return sc_k(table, idx2d)


def _tc_body(seq_ref, prev_ref, out_ref):
    del prev_ref
    out_ref[...] = seq_ref[...]


def _tc_fill_left(seq_t, out_t):
    b_total, l2, e_dim = out_t.shape
    l_dim = l2 // 2
    return pl.pallas_call(
        _tc_body,
        grid=(b_total // _BB,),
        in_specs=[
            pl.BlockSpec((_BB, l_dim, e_dim), lambda i: (i, 0, 0)),
            pl.BlockSpec(memory_space=pl.ANY),
        ],
        out_specs=pl.BlockSpec((_BB, l_dim, e_dim), lambda i: (i, 0, 0)),
        out_shape=jax.ShapeDtypeStruct((b_total, l2, e_dim), jnp.float32),
        input_output_aliases={1: 0},
    )(seq_t, out_t)


def kernel(sequence, time_index_sequence, variable_index_sequence, embed_table):
    del time_index_sequence
    b_total, e_dim, l_dim = sequence.shape
    seq_t = jnp.transpose(sequence, (0, 2, 1))
    idx2d = variable_index_sequence.reshape(
        2 * b_total, l_dim // 2).astype(jnp.int32)
    out_t = _sc_fill_right(embed_table, idx2d, b_total, l_dim)
    out_t = _tc_fill_left(seq_t, out_t)
    return jnp.transpose(out_t, (0, 2, 1))
